# baseline (speedup 1.0000x reference)
import functools

import jax
import jax.numpy as jnp
from jax import lax
from jax.experimental import pallas as pl
from jax.experimental.pallas import tpu as pltpu
from jax.experimental.pallas import tpu_sc as plsc


def _tc_masks_block(itemid_ref, labels_ref, masked_ref, target_ref, perm_ref,
                    enc_ref, *, seq_len: int):
    item = itemid_ref[...]
    bb = item.shape[0]

    nonpad = (item != 0).astype(jnp.int32)
    last = jnp.sum(nonpad, axis=1, keepdims=True) - 1
    last = jnp.where(last < 0, last + seq_len, last)

    col = jax.lax.broadcasted_iota(jnp.int32, (bb, seq_len), 1)
    is_last = col == last

    labels = jnp.where(is_last, item, 0)
    labels_ref[...] = labels
    masked_ref[...] = labels != 0

    masked_row = jnp.sum(jnp.where(is_last, jnp.minimum(jnp.abs(item), 1), 0),
                         axis=1, keepdims=True)
    enc_ref[...] = last + seq_len * masked_row

    i2 = jax.lax.broadcasted_iota(jnp.int32, (seq_len, seq_len), 0)
    j2 = jax.lax.broadcasted_iota(jnp.int32, (seq_len, seq_len), 1)
    target_ref[...] = jnp.broadcast_to((i2 == j2).astype(jnp.float32)[None],
                                       (bb, seq_len, seq_len))

    upper = j2 > i2
    perm = upper[None, :, :] | is_last[:, None, :]
    perm_ref[...] = perm.astype(jnp.int32)


def _tc_masks(itemid_seq, B, S):
    bb = 32
    return pl.pallas_call(
        functools.partial(_tc_masks_block, seq_len=S),
        grid=(B // bb,),
        in_specs=[pl.BlockSpec((bb, S), lambda i: (i, 0))],
        out_specs=[
            pl.BlockSpec((bb, S), lambda i: (i, 0)),
            pl.BlockSpec((bb, S), lambda i: (i, 0)),
            pl.BlockSpec((bb, S, S), lambda i: (i, 0, 0)),
            pl.BlockSpec((bb, S, S), lambda i: (i, 0, 0)),
            pl.BlockSpec((bb, 1), lambda i: (i, 0)),
        ],
        out_shape=[
            jax.ShapeDtypeStruct((B, S), itemid_seq.dtype),
            jax.ShapeDtypeStruct((B, S), jnp.bool_),
            jax.ShapeDtypeStruct((B, S, S), jnp.float32),
            jax.ShapeDtypeStruct((B, S, S), jnp.int32),
            jax.ShapeDtypeStruct((B, 1), jnp.int32),
        ],
        compiler_params=pltpu.CompilerParams(
            dimension_semantics=("parallel",),
        ),
    )(itemid_seq)


def _make_sc_copy(n):
    info = plsc.get_sparse_core_info()
    NC, NS = info.num_cores, info.num_subcores
    NW = NC * NS
    chunk = n // NW
    mesh = plsc.VectorSubcoreMesh(core_axis_name="c", subcore_axis_name="s")

    @functools.partial(
        pl.kernel, mesh=mesh,
        out_type=jax.ShapeDtypeStruct((n,), jnp.float32),
    )
    def sc_copy(src_hbm, out_hbm):
        wid = lax.axis_index("s") * NC + lax.axis_index("c")
        base = wid * chunk
        pltpu.sync_copy(src_hbm.at[pl.ds(base, chunk)],
                        out_hbm.at[pl.ds(base, chunk)])

    return sc_copy


def _fix_body(enc_ref, memb_ref, pos_ref, out_ref, *, seq_len: int):
    b = pl.program_id(0)
    out_ref[...] = pos_ref[...]

    @pl.when(enc_ref[b] >= seq_len)
    def _():
        out_ref[...] = memb_ref[...][None, None]


def _fix_rows(pos_copied, enc, memb, B, S, H):
    pos4 = pos_copied.reshape(B, S, 1, H)
    grid_spec = pltpu.PrefetchScalarGridSpec(
        num_scalar_prefetch=1,
        grid=(B,),
        in_specs=[
            pl.BlockSpec((1, H), lambda b, enc: (0, 0)),
            pl.BlockSpec((1, 1, 1, H), lambda b, enc: (b, enc[b] % S, 0, 0)),
        ],
        out_specs=pl.BlockSpec((1, 1, 1, H),
                               lambda b, enc: (b, enc[b] % S, 0, 0)),
    )
    out4 = pl.pallas_call(
        functools.partial(_fix_body, seq_len=S),
        grid_spec=grid_spec,
        out_shape=jax.ShapeDtypeStruct((B, S, 1, H), jnp.float32),
        input_output_aliases={2: 0},
    )(enc, memb, pos4)
    return out4.reshape(B, S, H)


def kernel(pos_emb, itemid_seq, training, masked_item_embedding):
    B, S, H = pos_emb.shape

    pos_flat = pos_emb.reshape(B * S * H)
    memb = masked_item_embedding.reshape(1, H).astype(pos_emb.dtype)

    pos_copied = _make_sc_copy(B * S * H)(pos_flat).reshape(B, S, H)

    labels, masked_labels, target_mapping, perm_mask_out, enc2 = _tc_masks(
        itemid_seq, B, S)
    enc = enc2.reshape(B)

    pos_emb_inp = _fix_rows(pos_copied, enc, memb, B, S, H)

    return (pos_emb_inp, labels, masked_labels, target_mapping, perm_mask_out)


# speedup vs baseline: 5.3884x; 5.3884x over previous
import functools

import jax
import jax.numpy as jnp
from jax.experimental import pallas as pl
from jax.experimental.pallas import tpu as pltpu


def _masks_block(item_ref, labels_ref, masked_ref, target_ref, perm_ref,
                 *, seq_len: int):
    item = item_ref[...]
    bb = item.shape[0]

    nonpad = (item != 0).astype(jnp.int32)
    last = jnp.sum(nonpad, axis=1, keepdims=True) - 1
    last = jnp.where(last < 0, last + seq_len, last)

    col = jax.lax.broadcasted_iota(jnp.int32, (bb, seq_len), 1)
    is_last = col == last

    labels = jnp.where(is_last, item, 0)
    labels_ref[...] = labels
    masked_ref[...] = labels != 0

    i2 = jax.lax.broadcasted_iota(jnp.int32, (seq_len, seq_len), 0)
    j2 = jax.lax.broadcasted_iota(jnp.int32, (seq_len, seq_len), 1)
    target_ref[...] = jnp.broadcast_to((i2 == j2).astype(jnp.float32)[None],
                                       (bb, seq_len, seq_len))

    upper = j2 > i2
    perm = upper[None, :, :] | is_last[:, None, :]
    perm_ref[...] = perm.astype(jnp.int32)


def _masks(itemid_seq, B, S):
    bb = 32
    return pl.pallas_call(
        functools.partial(_masks_block, seq_len=S),
        grid=(B // bb,),
        in_specs=[pl.BlockSpec((bb, S), lambda i: (i, 0))],
        out_specs=[
            pl.BlockSpec((bb, S), lambda i: (i, 0)),
            pl.BlockSpec((bb, S), lambda i: (i, 0)),
            pl.BlockSpec((bb, S, S), lambda i: (i, 0, 0)),
            pl.BlockSpec((bb, S, S), lambda i: (i, 0, 0)),
        ],
        out_shape=[
            jax.ShapeDtypeStruct((B, S), itemid_seq.dtype),
            jax.ShapeDtypeStruct((B, S), jnp.bool_),
            jax.ShapeDtypeStruct((B, S, S), jnp.float32),
            jax.ShapeDtypeStruct((B, S, S), jnp.int32),
        ],
        compiler_params=pltpu.CompilerParams(
            dimension_semantics=("parallel",),
        ),
    )(itemid_seq)


def _subst_block(mcol_ref, memb_ref, pos_ref, out_ref):
    rows, H = pos_ref.shape
    m = jnp.broadcast_to(mcol_ref[...], (rows, H))
    memb = jnp.broadcast_to(memb_ref[...], (rows, H))
    out_ref[...] = jnp.where(m != 0, memb, pos_ref[...])


def _substitute(pos2, mcol, memb, N, H):
    R = 2048
    return pl.pallas_call(
        _subst_block,
        grid=(N // R,),
        in_specs=[
            pl.BlockSpec((R, 1), lambda i: (i, 0)),
            pl.BlockSpec((1, H), lambda i: (0, 0)),
            pl.BlockSpec((R, H), lambda i: (i, 0)),
        ],
        out_specs=pl.BlockSpec((R, H), lambda i: (i, 0)),
        out_shape=jax.ShapeDtypeStruct((N, H), pos2.dtype),
        compiler_params=pltpu.CompilerParams(
            dimension_semantics=("parallel",),
        ),
    )(mcol, memb, pos2)


def kernel(pos_emb, itemid_seq, training, masked_item_embedding):
    B, S, H = pos_emb.shape

    labels, masked_labels, target_mapping, perm_mask_out = _masks(
        itemid_seq, B, S)

    mcol = masked_labels.reshape(B * S, 1).astype(jnp.float32)
    memb = masked_item_embedding.reshape(1, H).astype(pos_emb.dtype)
    pos2 = pos_emb.reshape(B * S, H)

    pos_emb_inp = _substitute(pos2, mcol, memb, B * S, H).reshape(B, S, H)

    return (pos_emb_inp, labels, masked_labels, target_mapping, perm_mask_out)


# speedup vs baseline: 6.2362x; 1.1574x over previous
import functools

import jax
import jax.numpy as jnp
from jax.experimental import pallas as pl
from jax.experimental.pallas import tpu as pltpu


def _fused_block(memb_ref, item_ref, pos_ref,
                 posout_ref, labels_ref, masked_ref, target_ref, perm_ref,
                 *, seq_len: int):
    item = item_ref[...]
    bb = item.shape[0]

    nonpad = (item != 0).astype(jnp.int32)
    last = jnp.sum(nonpad, axis=1, keepdims=True) - 1
    last = jnp.where(last < 0, last + seq_len, last)

    col = jax.lax.broadcasted_iota(jnp.int32, (bb, seq_len), 1)
    is_last = col == last

    labels = jnp.where(is_last, item, 0)
    labels_ref[...] = labels
    masked_ref[...] = labels != 0

    i2 = jax.lax.broadcasted_iota(jnp.int32, (seq_len, seq_len), 0)
    j2 = jax.lax.broadcasted_iota(jnp.int32, (seq_len, seq_len), 1)
    target_ref[...] = jnp.broadcast_to((i2 == j2).astype(jnp.float32)[None],
                                       (bb, seq_len, seq_len))

    upper = j2 > i2
    perm = upper[None, :, :] | is_last[:, None, :]
    perm_ref[...] = perm.astype(jnp.int32)

    g = jnp.sum(jnp.where(is_last, nonpad, 0), axis=1, keepdims=True)
    lastm = jnp.where(g > 0, last, -1)

    H = pos_ref.shape[-1]
    col3 = jax.lax.broadcasted_iota(jnp.int32, (bb, seq_len, H), 1)
    lastm3 = jnp.broadcast_to(lastm.reshape(bb, 1, 1), (bb, seq_len, H))
    memb3 = jnp.broadcast_to(memb_ref[...][None], (bb, seq_len, H))
    posout_ref[...] = jnp.where(col3 == lastm3, memb3, pos_ref[...])


def kernel(pos_emb, itemid_seq, training, masked_item_embedding):
    B, S, H = pos_emb.shape
    bb = 8

    memb = masked_item_embedding.reshape(1, H).astype(pos_emb.dtype)

    outs = pl.pallas_call(
        functools.partial(_fused_block, seq_len=S),
        grid=(B // bb,),
        in_specs=[
            pl.BlockSpec((1, H), lambda i: (0, 0)),
            pl.BlockSpec((bb, S), lambda i: (i, 0)),
            pl.BlockSpec((bb, S, H), lambda i: (i, 0, 0)),
        ],
        out_specs=[
            pl.BlockSpec((bb, S, H), lambda i: (i, 0, 0)),
            pl.BlockSpec((bb, S), lambda i: (i, 0)),
            pl.BlockSpec((bb, S), lambda i: (i, 0)),
            pl.BlockSpec((bb, S, S), lambda i: (i, 0, 0)),
            pl.BlockSpec((bb, S, S), lambda i: (i, 0, 0)),
        ],
        out_shape=[
            jax.ShapeDtypeStruct((B, S, H), pos_emb.dtype),
            jax.ShapeDtypeStruct((B, S), itemid_seq.dtype),
            jax.ShapeDtypeStruct((B, S), jnp.bool_),
            jax.ShapeDtypeStruct((B, S, S), jnp.float32),
            jax.ShapeDtypeStruct((B, S, S), jnp.int32),
        ],
        compiler_params=pltpu.CompilerParams(
            dimension_semantics=("parallel",),
        ),
    )(memb, itemid_seq, pos_emb)

    pos_emb_inp, labels, masked_labels, target_mapping, perm_mask_out = outs
    return (pos_emb_inp, labels, masked_labels, target_mapping, perm_mask_out)


# speedup vs baseline: 6.3748x; 1.0222x over previous
import functools

import jax
import jax.numpy as jnp
from jax.experimental import pallas as pl
from jax.experimental.pallas import tpu as pltpu


def _fused_block(memb_ref, item_ref, pos_ref,
                 posout_ref, labels_ref, masked_ref, target_ref, perm_ref,
                 *, seq_len: int):
    item = item_ref[...]
    bb = item.shape[0]

    nonpad = (item != 0).astype(jnp.int32)
    last = jnp.sum(nonpad, axis=1, keepdims=True) - 1
    last = jnp.where(last < 0, last + seq_len, last)

    col = jax.lax.broadcasted_iota(jnp.int32, (bb, seq_len), 1)
    is_last = col == last

    labels = jnp.where(is_last, item, 0)
    labels_ref[...] = labels
    masked_ref[...] = labels != 0

    i2 = jax.lax.broadcasted_iota(jnp.int32, (seq_len, seq_len), 0)
    j2 = jax.lax.broadcasted_iota(jnp.int32, (seq_len, seq_len), 1)
    target_ref[...] = jnp.broadcast_to((i2 == j2).astype(jnp.float32)[None],
                                       (bb, seq_len, seq_len))

    upper = j2 > i2
    perm = upper[None, :, :] | is_last[:, None, :]
    perm_ref[...] = perm.astype(jnp.int32)

    g = jnp.sum(jnp.where(is_last, nonpad, 0), axis=1, keepdims=True)
    lastm = jnp.where(g > 0, last, -1)

    H = pos_ref.shape[-1]
    col3 = jax.lax.broadcasted_iota(jnp.int32, (bb, seq_len, H), 1)
    lastm3 = jnp.broadcast_to(lastm.reshape(bb, 1, 1), (bb, seq_len, H))
    memb3 = jnp.broadcast_to(memb_ref[...][None], (bb, seq_len, H))
    posout_ref[...] = jnp.where(col3 == lastm3, memb3, pos_ref[...])


def kernel(pos_emb, itemid_seq, training, masked_item_embedding):
    B, S, H = pos_emb.shape
    bb = 16

    memb = masked_item_embedding.reshape(1, H).astype(pos_emb.dtype)

    outs = pl.pallas_call(
        functools.partial(_fused_block, seq_len=S),
        grid=(B // bb,),
        in_specs=[
            pl.BlockSpec((1, H), lambda i: (0, 0)),
            pl.BlockSpec((bb, S), lambda i: (i, 0)),
            pl.BlockSpec((bb, S, H), lambda i: (i, 0, 0)),
        ],
        out_specs=[
            pl.BlockSpec((bb, S, H), lambda i: (i, 0, 0)),
            pl.BlockSpec((bb, S), lambda i: (i, 0)),
            pl.BlockSpec((bb, S), lambda i: (i, 0)),
            pl.BlockSpec((bb, S, S), lambda i: (i, 0, 0)),
            pl.BlockSpec((bb, S, S), lambda i: (i, 0, 0)),
        ],
        out_shape=[
            jax.ShapeDtypeStruct((B, S, H), pos_emb.dtype),
            jax.ShapeDtypeStruct((B, S), itemid_seq.dtype),
            jax.ShapeDtypeStruct((B, S), jnp.bool_),
            jax.ShapeDtypeStruct((B, S, S), jnp.float32),
            jax.ShapeDtypeStruct((B, S, S), jnp.int32),
        ],
        compiler_params=pltpu.CompilerParams(
            dimension_semantics=("parallel",),
        ),
    )(memb, itemid_seq, pos_emb)

    pos_emb_inp, labels, masked_labels, target_mapping, perm_mask_out = outs
    return (pos_emb_inp, labels, masked_labels, target_mapping, perm_mask_out)


# speedup vs baseline: 6.4359x; 1.0096x over previous
import functools

import jax
import jax.numpy as jnp
from jax.experimental import pallas as pl
from jax.experimental.pallas import tpu as pltpu


def _fused_block(memb_ref, item_ref, pos_ref,
                 posout_ref, labels_ref, masked_ref, target_ref, perm_ref,
                 *, seq_len: int):
    item = item_ref[...]
    bb = item.shape[0]

    nonpad = (item != 0).astype(jnp.int32)
    last = jnp.sum(nonpad, axis=1, keepdims=True) - 1
    last = jnp.where(last < 0, last + seq_len, last)

    col = jax.lax.broadcasted_iota(jnp.int32, (bb, seq_len), 1)
    is_last = col == last

    labels = jnp.where(is_last, item, 0)
    labels_ref[...] = labels
    masked_ref[...] = labels != 0

    i2 = jax.lax.broadcasted_iota(jnp.int32, (seq_len, seq_len), 0)
    j2 = jax.lax.broadcasted_iota(jnp.int32, (seq_len, seq_len), 1)
    target_ref[...] = jnp.broadcast_to((i2 == j2).astype(jnp.float32)[None],
                                       (bb, seq_len, seq_len))

    upper = j2 > i2
    perm = upper[None, :, :] | is_last[:, None, :]
    perm_ref[...] = perm.astype(jnp.int32)

    g = jnp.sum(jnp.where(is_last, nonpad, 0), axis=1, keepdims=True)
    lastm = jnp.where(g > 0, last, -1)

    H = pos_ref.shape[-1]
    col3 = jax.lax.broadcasted_iota(jnp.int32, (bb, seq_len, H), 1)
    lastm3 = jnp.broadcast_to(lastm.reshape(bb, 1, 1), (bb, seq_len, H))
    memb3 = jnp.broadcast_to(memb_ref[...][None], (bb, seq_len, H))
    posout_ref[...] = jnp.where(col3 == lastm3, memb3, pos_ref[...])


def kernel(pos_emb, itemid_seq, training, masked_item_embedding):
    B, S, H = pos_emb.shape
    bb = 32

    memb = masked_item_embedding.reshape(1, H).astype(pos_emb.dtype)

    outs = pl.pallas_call(
        functools.partial(_fused_block, seq_len=S),
        grid=(B // bb,),
        in_specs=[
            pl.BlockSpec((1, H), lambda i: (0, 0)),
            pl.BlockSpec((bb, S), lambda i: (i, 0)),
            pl.BlockSpec((bb, S, H), lambda i: (i, 0, 0)),
        ],
        out_specs=[
            pl.BlockSpec((bb, S, H), lambda i: (i, 0, 0)),
            pl.BlockSpec((bb, S), lambda i: (i, 0)),
            pl.BlockSpec((bb, S), lambda i: (i, 0)),
            pl.BlockSpec((bb, S, S), lambda i: (i, 0, 0)),
            pl.BlockSpec((bb, S, S), lambda i: (i, 0, 0)),
        ],
        out_shape=[
            jax.ShapeDtypeStruct((B, S, H), pos_emb.dtype),
            jax.ShapeDtypeStruct((B, S), itemid_seq.dtype),
            jax.ShapeDtypeStruct((B, S), jnp.bool_),
            jax.ShapeDtypeStruct((B, S, S), jnp.float32),
            jax.ShapeDtypeStruct((B, S, S), jnp.int32),
        ],
        compiler_params=pltpu.CompilerParams(
            dimension_semantics=("parallel",),
        ),
    )(memb, itemid_seq, pos_emb)

    pos_emb_inp, labels, masked_labels, target_mapping, perm_mask_out = outs
    return (pos_emb_inp, labels, masked_labels, target_mapping, perm_mask_out)


# speedup vs baseline: 6.4673x; 1.0049x over previous
import functools

import jax
import jax.numpy as jnp
from jax.experimental import pallas as pl
from jax.experimental.pallas import tpu as pltpu


def _fused_block(memb_ref, item_ref, pos_ref,
                 posout_ref, labels_ref, masked_ref, target_ref, perm_ref,
                 *, seq_len: int):
    item = item_ref[...]
    bb = item.shape[0]

    nonpad = (item != 0).astype(jnp.int32)
    last = jnp.sum(nonpad, axis=1, keepdims=True) - 1
    last = jnp.where(last < 0, last + seq_len, last)

    col = jax.lax.broadcasted_iota(jnp.int32, (bb, seq_len), 1)
    is_last = col == last

    labels = jnp.where(is_last, item, 0)
    labels_ref[...] = labels
    masked_ref[...] = labels != 0

    i2 = jax.lax.broadcasted_iota(jnp.int32, (seq_len, seq_len), 0)
    j2 = jax.lax.broadcasted_iota(jnp.int32, (seq_len, seq_len), 1)
    target_ref[...] = jnp.broadcast_to((i2 == j2).astype(jnp.float32)[None],
                                       (bb, seq_len, seq_len))

    upper_i32 = (j2 > i2).astype(jnp.int32)
    perm_ref[...] = jnp.maximum(upper_i32[None, :, :],
                                is_last.astype(jnp.int32)[:, None, :])

    g = jnp.sum(jnp.where(is_last, nonpad, 0), axis=1, keepdims=True)
    lastm = jnp.where(g > 0, last, -1)

    H = pos_ref.shape[-1]
    col3 = jax.lax.broadcasted_iota(jnp.int32, (bb, seq_len, H), 1)
    lastm3 = jnp.broadcast_to(lastm.reshape(bb, 1, 1), (bb, seq_len, H))
    memb3 = jnp.broadcast_to(memb_ref[...][None], (bb, seq_len, H))
    posout_ref[...] = jnp.where(col3 == lastm3, memb3, pos_ref[...])


def kernel(pos_emb, itemid_seq, training, masked_item_embedding):
    B, S, H = pos_emb.shape
    bb = 32

    memb = masked_item_embedding.reshape(1, H).astype(pos_emb.dtype)

    outs = pl.pallas_call(
        functools.partial(_fused_block, seq_len=S),
        grid=(B // bb,),
        in_specs=[
            pl.BlockSpec((1, H), lambda i: (0, 0)),
            pl.BlockSpec((bb, S), lambda i: (i, 0)),
            pl.BlockSpec((bb, S, H), lambda i: (i, 0, 0)),
        ],
        out_specs=[
            pl.BlockSpec((bb, S, H), lambda i: (i, 0, 0)),
            pl.BlockSpec((bb, S), lambda i: (i, 0)),
            pl.BlockSpec((bb, S), lambda i: (i, 0)),
            pl.BlockSpec((bb, S, S), lambda i: (i, 0, 0)),
            pl.BlockSpec((bb, S, S), lambda i: (i, 0, 0)),
        ],
        out_shape=[
            jax.ShapeDtypeStruct((B, S, H), pos_emb.dtype),
            jax.ShapeDtypeStruct((B, S), itemid_seq.dtype),
            jax.ShapeDtypeStruct((B, S), jnp.bool_),
            jax.ShapeDtypeStruct((B, S, S), jnp.float32),
            jax.ShapeDtypeStruct((B, S, S), jnp.int32),
        ],
        compiler_params=pltpu.CompilerParams(
            dimension_semantics=("parallel",),
        ),
    )(memb, itemid_seq, pos_emb)

    pos_emb_inp, labels, masked_labels, target_mapping, perm_mask_out = outs
    return (pos_emb_inp, labels, masked_labels, target_mapping, perm_mask_out)
